# Initial kernel scaffold; baseline (speedup 1.0000x reference)
import jax
import jax.numpy as jnp
from jax.experimental import pallas as pl


def kernel(x, edge_index, W1, b1, W2, b2, lin_W, b_lin):
    raise NotImplementedError("write your pallas kernel here")


# speedup vs baseline: 14.7289x; 14.7289x over previous
import functools

import jax
import jax.numpy as jnp
from jax import lax
from jax.experimental import pallas as pl
from jax.experimental.pallas import tpu as pltpu
from jax.experimental.pallas import tpu_sc as plsc

N = 10000
E = 320000
IN_DIM = 128
D = 64
NC = 2
NS = 16
NW = NC * NS
B = 128
NB = 80
EPAD = NW * NB * B
NPAD = 10240
RPS = NPAD // NS

_MESH = plsc.VectorSubcoreMesh(core_axis_name="c", subcore_axis_name="s")


@functools.partial(
    pl.kernel,
    out_type=jax.ShapeDtypeStruct((NC, NPAD), jnp.float32),
    mesh=_MESH,
    scratch_types=[
        pltpu.VMEM_SHARED((NPAD,), jnp.float32),
        pltpu.VMEM((NB, B), jnp.int32),
        pltpu.VMEM((B,), jnp.float32),
    ],
)
def _deg(dst_hbm, zeros1_hbm, out_hbm, acc, didx, ones_v):
    c = lax.axis_index("c")
    s = lax.axis_index("s")
    wid = s * NC + c
    pltpu.sync_copy(zeros1_hbm.at[pl.ds(s * RPS, RPS)], acc.at[pl.ds(s * RPS, RPS)])
    for i in range(B // 16):
        ones_v[pl.ds(i * 16, 16)] = jnp.ones((16,), jnp.float32)
    pltpu.sync_copy(dst_hbm.at[wid], didx)
    plsc.subcore_barrier()

    def body(j, carry):
        pltpu.sync_copy(ones_v, acc.at[didx.at[j]], add=True)
        return carry

    lax.fori_loop(0, NB, body, 0)
    plsc.subcore_barrier()
    pltpu.sync_copy(acc.at[pl.ds(s * RPS, RPS)], out_hbm.at[c, pl.ds(s * RPS, RPS)])


@functools.partial(
    pl.kernel,
    out_type=jax.ShapeDtypeStruct((NC, NPAD, D), jnp.float32),
    mesh=_MESH,
    scratch_types=[
        pltpu.VMEM_SHARED((NPAD, D), jnp.float32),
        pltpu.VMEM((NB, B), jnp.int32),
        pltpu.VMEM((NB, B), jnp.int32),
        pltpu.VMEM((B, D), jnp.float32),
        pltpu.SemaphoreType.DMA,
    ],
    compiler_params=pltpu.CompilerParams(use_tc_tiling_on_sc=False),
)
def _prop(tbl_hbm, src_hbm, dst_hbm, zeros2_hbm, out_hbm, acc, sidx, didx, rows, sem):
    c = lax.axis_index("c")
    s = lax.axis_index("s")
    wid = s * NC + c
    pltpu.sync_copy(zeros2_hbm.at[pl.ds(s * RPS, RPS)], acc.at[pl.ds(s * RPS, RPS)])
    pltpu.sync_copy(src_hbm.at[wid], sidx)
    pltpu.sync_copy(dst_hbm.at[wid], didx)
    plsc.subcore_barrier()

    def body(j, carry):
        pltpu.async_copy(tbl_hbm.at[sidx.at[j]], rows, sem).wait()
        pltpu.sync_copy(rows, acc.at[didx.at[j]], add=True)
        return carry

    lax.fori_loop(0, NB, body, 0)
    plsc.subcore_barrier()
    pltpu.sync_copy(acc.at[pl.ds(s * RPS, RPS)], out_hbm.at[c, pl.ds(s * RPS, RPS)])


def _tc1_body(x_ref, w_ref, dinv_ref, o_ref):
    h = jnp.dot(x_ref[...], w_ref[...], preferred_element_type=jnp.float32)
    o_ref[...] = h * dinv_ref[...]


def _tc_mid_body(p_ref, hp_ref, dinv_ref, b_ref, w_ref, o_ref):
    p = p_ref[0, :N, :] + p_ref[1, :N, :]
    z = jnp.maximum((p + hp_ref[...]) * dinv_ref[...] + b_ref[...], 0.0)
    o_ref[...] = jnp.dot(z, w_ref[...], preferred_element_type=jnp.float32) * dinv_ref[...]


def _tc_fin_body(p_ref, hp_ref, dinv_ref, b_ref, w_ref, blin_ref, o_ref):
    p = p_ref[0, :N, :] + p_ref[1, :N, :]
    z = jnp.maximum((p + hp_ref[...]) * dinv_ref[...] + b_ref[...], 0.0)
    o_ref[...] = jnp.dot(z, w_ref[...], preferred_element_type=jnp.float32) + blin_ref[...]


def kernel(x, edge_index, W1, b1, W2, b2, lin_W, b_lin):
    f32 = jnp.float32
    ei = edge_index.astype(jnp.int32)
    pad = EPAD - E
    srcp = jnp.concatenate([ei[0], jnp.zeros((pad,), jnp.int32)]).reshape(NW, NB, B)
    dstp = jnp.concatenate([ei[1], jnp.full((pad,), N, jnp.int32)]).reshape(NW, NB, B)
    zeros1 = jnp.zeros((NPAD,), f32)
    zeros2 = jnp.zeros((NPAD, D), f32)

    degp = _deg(dstp, zeros1)
    deg = degp[0, :N] + degp[1, :N] + 1.0
    dinv = lax.rsqrt(deg)[:, None]

    h1p = pl.pallas_call(
        _tc1_body, out_shape=jax.ShapeDtypeStruct((N, D), f32),
    )(x, W1, dinv)

    p1 = _prop(h1p, srcp, dstp, zeros2)

    h2p = pl.pallas_call(
        _tc_mid_body, out_shape=jax.ShapeDtypeStruct((N, D), f32),
    )(p1, h1p, dinv, b1.reshape(1, D), W2)

    p2 = _prop(h2p, srcp, dstp, zeros2)

    logits = pl.pallas_call(
        _tc_fin_body, out_shape=jax.ShapeDtypeStruct((N, 2), f32),
    )(p2, h2p, dinv, b2.reshape(1, D), lin_W, b_lin.reshape(1, 2))
    return logits


# speedup vs baseline: 17.2490x; 1.1711x over previous
import functools

import jax
import jax.numpy as jnp
from jax import lax
from jax.experimental import pallas as pl
from jax.experimental.pallas import tpu as pltpu
from jax.experimental.pallas import tpu_sc as plsc

N = 10000
E = 320000
IN_DIM = 128
D = 64
NC = 2
NS = 16
NW = NC * NS
B = 128
NB = 80
EPAD = NW * NB * B
NPAD = 10240
RPS = NPAD // NS

_MESH = plsc.VectorSubcoreMesh(core_axis_name="c", subcore_axis_name="s")


@functools.partial(
    pl.kernel,
    out_type=jax.ShapeDtypeStruct((NC, NPAD), jnp.float32),
    mesh=_MESH,
    scratch_types=[
        pltpu.VMEM_SHARED((NPAD,), jnp.float32),
        pltpu.VMEM((NB, B), jnp.int32),
        pltpu.VMEM((B,), jnp.float32),
    ],
)
def _deg(dst_hbm, zeros1_hbm, out_hbm, acc, didx, ones_v):
    c = lax.axis_index("c")
    s = lax.axis_index("s")
    wid = s * NC + c
    pltpu.sync_copy(zeros1_hbm.at[pl.ds(s * RPS, RPS)], acc.at[pl.ds(s * RPS, RPS)])
    for i in range(B // 16):
        ones_v[pl.ds(i * 16, 16)] = jnp.ones((16,), jnp.float32)
    pltpu.sync_copy(dst_hbm.at[wid], didx)
    plsc.subcore_barrier()

    def body(j, carry):
        pltpu.sync_copy(ones_v, acc.at[didx.at[j]], add=True)
        return carry

    lax.fori_loop(0, NB, body, 0)
    plsc.subcore_barrier()
    pltpu.sync_copy(acc.at[pl.ds(s * RPS, RPS)], out_hbm.at[c, pl.ds(s * RPS, RPS)])


@functools.partial(
    pl.kernel,
    out_type=jax.ShapeDtypeStruct((NC, NPAD, D), jnp.float32),
    mesh=_MESH,
    scratch_types=[
        pltpu.VMEM_SHARED((NPAD, D), jnp.float32),
        pltpu.VMEM((NB, B), jnp.int32),
        pltpu.VMEM((NB, B), jnp.int32),
        pltpu.VMEM((B, D), jnp.float32),
        pltpu.VMEM((B, D), jnp.float32),
        pltpu.SemaphoreType.DMA,
        pltpu.SemaphoreType.DMA,
    ],
    compiler_params=pltpu.CompilerParams(use_tc_tiling_on_sc=False),
)
def _prop(tbl_hbm, src_hbm, dst_hbm, zeros2_hbm, out_hbm, acc, sidx, didx,
          rows_a, rows_b, sem_a, sem_b):
    c = lax.axis_index("c")
    s = lax.axis_index("s")
    wid = s * NC + c
    pltpu.async_copy(src_hbm.at[wid], sidx, sem_a)
    pltpu.async_copy(dst_hbm.at[wid], didx, sem_b)
    pltpu.sync_copy(zeros2_hbm.at[pl.ds(s * RPS, RPS)], acc.at[pl.ds(s * RPS, RPS)])
    pltpu.make_async_copy(src_hbm.at[wid], sidx, sem_a).wait()
    pltpu.make_async_copy(dst_hbm.at[wid], didx, sem_b).wait()
    plsc.subcore_barrier()
    pltpu.async_copy(tbl_hbm.at[sidx.at[0]], rows_a, sem_a)

    def body(jj, carry):
        j0 = jj * 2
        pltpu.async_copy(tbl_hbm.at[sidx.at[j0 + 1]], rows_b, sem_b)
        pltpu.make_async_copy(tbl_hbm.at[sidx.at[j0]], rows_a, sem_a).wait()
        pltpu.sync_copy(rows_a, acc.at[didx.at[j0]], add=True)

        @pl.when(jj < NB // 2 - 1)
        def _():
            pltpu.async_copy(tbl_hbm.at[sidx.at[j0 + 2]], rows_a, sem_a)

        pltpu.make_async_copy(tbl_hbm.at[sidx.at[j0 + 1]], rows_b, sem_b).wait()
        pltpu.sync_copy(rows_b, acc.at[didx.at[j0 + 1]], add=True)
        return carry

    lax.fori_loop(0, NB // 2, body, 0)
    plsc.subcore_barrier()
    pltpu.sync_copy(acc.at[pl.ds(s * RPS, RPS)], out_hbm.at[c, pl.ds(s * RPS, RPS)])


def _tc1_body(x_ref, w_ref, dinv_ref, o_ref):
    h = jnp.dot(x_ref[...], w_ref[...], preferred_element_type=jnp.float32)
    o_ref[...] = h * dinv_ref[...]


def _tc_mid_body(p_ref, hp_ref, dinv_ref, b_ref, w_ref, o_ref):
    p = p_ref[0, :N, :] + p_ref[1, :N, :]
    z = jnp.maximum((p + hp_ref[...]) * dinv_ref[...] + b_ref[...], 0.0)
    o_ref[...] = jnp.dot(z, w_ref[...], preferred_element_type=jnp.float32) * dinv_ref[...]


def _tc_fin_body(p_ref, hp_ref, dinv_ref, b_ref, w_ref, blin_ref, o_ref):
    p = p_ref[0, :N, :] + p_ref[1, :N, :]
    z = jnp.maximum((p + hp_ref[...]) * dinv_ref[...] + b_ref[...], 0.0)
    o_ref[...] = jnp.dot(z, w_ref[...], preferred_element_type=jnp.float32) + blin_ref[...]


def kernel(x, edge_index, W1, b1, W2, b2, lin_W, b_lin):
    f32 = jnp.float32
    ei = edge_index.astype(jnp.int32)
    pad = EPAD - E
    srcp = jnp.concatenate([ei[0], jnp.zeros((pad,), jnp.int32)]).reshape(NW, NB, B)
    dstp = jnp.concatenate([ei[1], jnp.full((pad,), N, jnp.int32)]).reshape(NW, NB, B)
    zeros1 = jnp.zeros((NPAD,), f32)
    zeros2 = jnp.zeros((NPAD, D), f32)

    degp = _deg(dstp, zeros1)
    deg = degp[0, :N] + degp[1, :N] + 1.0
    dinv = lax.rsqrt(deg)[:, None]

    h1p = pl.pallas_call(
        _tc1_body, out_shape=jax.ShapeDtypeStruct((N, D), f32),
    )(x, W1, dinv)

    p1 = _prop(h1p, srcp, dstp, zeros2)

    h2p = pl.pallas_call(
        _tc_mid_body, out_shape=jax.ShapeDtypeStruct((N, D), f32),
    )(p1, h1p, dinv, b1.reshape(1, D), W2)

    p2 = _prop(h2p, srcp, dstp, zeros2)

    logits = pl.pallas_call(
        _tc_fin_body, out_shape=jax.ShapeDtypeStruct((N, 2), f32),
    )(p2, h2p, dinv, b2.reshape(1, D), lin_W, b_lin.reshape(1, 2))
    return logits
